# baseline (speedup 1.0000x reference)
import functools

import jax
import jax.numpy as jnp
from jax import lax
from jax.experimental import pallas as pl
from jax.experimental.pallas import tpu as pltpu
from jax.experimental.pallas import tpu_sc as plsc

_GRP = 4


def _sc_gather_pair(in_w, out_w, target, context):
    D, V = in_w.shape
    B = target.shape[0]
    info = plsc.get_sparse_core_info()
    NC, NS = info.num_cores, info.num_subcores
    NW = NC * NS
    b_per_w = B // NW
    mesh = plsc.VectorSubcoreMesh(core_axis_name="c", subcore_axis_name="s")

    @functools.partial(
        pl.kernel,
        out_type=(
            jax.ShapeDtypeStruct((B * D,), jnp.float32),
            jax.ShapeDtypeStruct((B * D,), jnp.float32),
        ),
        mesh=mesh,
        compiler_params=pltpu.CompilerParams(needs_layout_passes=False),
        scratch_types=[
            pltpu.VMEM((b_per_w,), jnp.int32),
            pltpu.SMEM((2 * b_per_w,), jnp.int32),
            pltpu.VMEM((2 * _GRP * D, 128), jnp.float32),
            pltpu.VMEM((b_per_w * D,), jnp.float32),
            pltpu.SemaphoreType.DMA,
            pltpu.SemaphoreType.DMA,
        ],
    )
    def gather_k(wt_a, wt_b, tgt_hbm, ctx_hbm, out_a, out_b,
                 idx_v, idx_s, pnl, rows, sem0, sem1):
        wid = lax.axis_index("s") * NC + lax.axis_index("c")
        base = wid * b_per_w
        lane = lax.iota(jnp.int32, 16)

        for t, src in ((0, tgt_hbm), (1, ctx_hbm)):
            pltpu.sync_copy(src.at[pl.ds(base, b_per_w)], idx_v)
            for j in range(b_per_w // 16):
                va = idx_v[pl.ds(j * 16, 16)]
                for l in range(16):
                    idx_s[t * b_per_w + j * 16 + l] = jnp.sum(
                        jnp.where(lane == l, va, 0))

        def run_table(t, wt, out_hbm):
            def fire(i, slot, sem):
                v = idx_s[t * b_per_w + i]
                off = pl.multiple_of((v // 128) * 128, 128)
                pltpu.async_copy(wt.at[:, pl.ds(off, 128)],
                                 pnl.at[pl.ds(slot * D, D), :], sem)

            def extract(i, slot):
                v = idx_s[t * b_per_w + i]
                r = lax.rem(v, 128)
                coli = jnp.full((16,), r, jnp.int32)
                for c4 in range(D // 16):
                    rowi = slot * D + c4 * 16 + lane
                    vals = plsc.load_gather(pnl, [rowi, coli])
                    rows[pl.ds(i * D + c4 * 16, 16)] = vals

            n_grp = b_per_w // _GRP
            n_pairs = n_grp // 2
            for s in range(_GRP):
                fire(s, s, sem0)
            for s in range(_GRP):
                fire(_GRP + s, _GRP + s, sem1)

            def body(p, carry):
                for half, sem in ((0, sem0), (1, sem1)):
                    g = 2 * p + half
                    for s in range(_GRP):
                        pltpu.make_async_copy(
                            wt.at[:, pl.ds(0, 128)],
                            pnl.at[pl.ds(0, D), :], sem).wait()
                    for s in range(_GRP):
                        extract(g * _GRP + s, half * _GRP + s)

                    @pl.when(p < n_pairs - 1)
                    def _():
                        for s in range(_GRP):
                            fire((g + 2) * _GRP + s, half * _GRP + s, sem)
                return carry

            lax.fori_loop(0, n_pairs, body, 0)
            pltpu.sync_copy(rows, out_hbm.at[pl.ds(base * D, b_per_w * D)])

        run_table(0, wt_a, out_a)
        run_table(1, wt_b, out_b)

    return gather_k(in_w, out_w, target, context)


_TCHUNK = 128


def _tc_gather(wt, idx):
    D = wt.shape[0]
    T = idx.shape[0]
    n_steps = T // _TCHUNK

    def body(idx_ref, wt_ref, o_ref, pnl, sem_a, sem_b):
        k = pl.program_id(0)
        lanes = lax.broadcasted_iota(jnp.int32, (8, 128), 1)
        half_sz = _TCHUNK * D

        def fire(chunk, half, sem):
            for j in range(_TCHUNK):
                v = idx_ref[chunk * _TCHUNK + j]
                off = pl.multiple_of((v // 128) * 128, 128)
                pltpu.make_async_copy(
                    wt_ref.at[:, pl.ds(off, 128)],
                    pnl.at[pl.ds(half * half_sz + j * D, D), :],
                    sem).start()

        def drain_extract(chunk, half, sem):
            pltpu.make_async_copy(
                wt_ref.at[:, pl.ds(0, 128 * _TCHUNK)],
                pnl.at[pl.ds(half * half_sz, half_sz), :], sem).wait()
            for b in range(_TCHUNK // 8):
                rs = [idx_ref[chunk * _TCHUNK + b * 8 + p] % 128
                      for p in range(8)]
                rcol = jnp.concatenate(
                    [jnp.full((1, 128), r, jnp.int32) for r in rs], axis=0)
                oh = (lanes == rcol).astype(jnp.float32)
                pstack = pnl[pl.ds(half * half_sz + b * 8 * D, 8 * D), :]
                m2 = lax.dot_general(oh, pstack, (((1,), (1,)), ((), ())),
                                     preferred_element_type=jnp.float32)
                rows8 = jnp.concatenate(
                    [lax.slice(m2, (p, p * D), (p + 1, p * D + D))
                     for p in range(8)], axis=0)
                o_ref[pl.ds(b * 8, 8), :] = rows8

        even = lax.rem(k, 2) == 0

        @pl.when(k == 0)
        def _():
            fire(0, 0, sem_a)

        @pl.when(jnp.logical_and(k + 1 < n_steps, even))
        def _():
            fire(k + 1, 1, sem_b)

        @pl.when(jnp.logical_and(k + 1 < n_steps,
                                 jnp.logical_not(even)))
        def _():
            fire(k + 1, 0, sem_a)

        @pl.when(even)
        def _():
            drain_extract(k, 0, sem_a)

        @pl.when(jnp.logical_not(even))
        def _():
            drain_extract(k, 1, sem_b)

    return pl.pallas_call(
        body,
        grid_spec=pltpu.PrefetchScalarGridSpec(
            num_scalar_prefetch=1,
            grid=(n_steps,),
            in_specs=[pl.BlockSpec(memory_space=pl.ANY)],
            out_specs=pl.BlockSpec((_TCHUNK, D), lambda k, idx: (k, 0)),
            scratch_shapes=[
                pltpu.VMEM((2 * _TCHUNK * D, 128), jnp.float32),
                pltpu.SemaphoreType.DMA,
                pltpu.SemaphoreType.DMA,
            ],
        ),
        out_shape=jax.ShapeDtypeStruct((T, D), jnp.float32),
    )(idx, wt)


def _tc_score_split(a_sc, a_tc, b_sc, b_tc):
    s, D = a_sc.shape
    t = a_tc.shape[0]
    B = s + t
    BLK = 512
    n_sc = s // BLK

    def matmul_body(a_ref, at_ref, b_ref, bt_ref, o_ref):
        i = pl.program_id(0)
        dims = (((1,), (1,)), ((), ()))
        a = jnp.where(i < n_sc, a_ref[...], at_ref[...])
        o_ref[:, pl.ds(0, s)] = lax.dot_general(
            a, b_ref[...], dims, preferred_element_type=jnp.float32)
        o_ref[:, pl.ds(s, t)] = lax.dot_general(
            a, bt_ref[...], dims, preferred_element_type=jnp.float32)

    return pl.pallas_call(
        matmul_body,
        grid=(B // BLK,),
        in_specs=[
            pl.BlockSpec((BLK, D), lambda i: (jnp.minimum(i, n_sc - 1), 0)),
            pl.BlockSpec((BLK, D),
                         lambda i: (jnp.maximum(i - n_sc, 0), 0)),
            pl.BlockSpec((s, D), lambda i: (0, 0)),
            pl.BlockSpec((t, D), lambda i: (0, 0)),
        ],
        out_specs=pl.BlockSpec((BLK, B), lambda i: (i, 0)),
        out_shape=jax.ShapeDtypeStruct((B, B), jnp.float32),
    )(a_sc, a_tc, b_sc, b_tc)


_SC_SHARE = 3072


def kernel(target, context, in_embed_w, out_embed_w):
    target = target.astype(jnp.int32)
    context = context.astype(jnp.int32)
    D = in_embed_w.shape[1]
    wt_a = in_embed_w.T
    wt_b = out_embed_w.T
    s = _SC_SHARE
    a_flat, b_flat = _sc_gather_pair(wt_a, wt_b, target[:s], context[:s])
    a_tc = _tc_gather(wt_a, target[s:])
    b_tc = _tc_gather(wt_b, context[s:])
    return _tc_score_split(a_flat.reshape(s, D), a_tc,
                           b_flat.reshape(s, D), b_tc)


# speedup vs baseline: 1.0998x; 1.0998x over previous
import functools

import jax
import jax.numpy as jnp
from jax import lax
from jax.experimental import pallas as pl
from jax.experimental.pallas import tpu as pltpu
from jax.experimental.pallas import tpu_sc as plsc

_GRP = 4


def _sc_gather_pair(in_w, out_w, target, context, B):
    D, V = in_w.shape
    info = plsc.get_sparse_core_info()
    NC, NS = info.num_cores, info.num_subcores
    NW = NC * NS
    b_per_w = B // NW
    mesh = plsc.VectorSubcoreMesh(core_axis_name="c", subcore_axis_name="s")

    @functools.partial(
        pl.kernel,
        out_type=(
            jax.ShapeDtypeStruct((B * 128,), jnp.float32),
            jax.ShapeDtypeStruct((B * 128,), jnp.float32),
        ),
        mesh=mesh,
        compiler_params=pltpu.CompilerParams(needs_layout_passes=False),
        scratch_types=[
            pltpu.VMEM((b_per_w,), jnp.int32),
            pltpu.SMEM((2 * b_per_w,), jnp.int32),
            pltpu.VMEM((2 * _GRP * D, 128), jnp.float32),
            pltpu.VMEM((b_per_w * 128,), jnp.float32),
            pltpu.SemaphoreType.DMA,
            pltpu.SemaphoreType.DMA,
        ],
    )
    def gather_k(wt_a, wt_b, tgt_hbm, ctx_hbm, out_a, out_b,
                 idx_v, idx_s, pnl, rows, sem0, sem1):
        wid = lax.axis_index("s") * NC + lax.axis_index("c")
        base = wid * b_per_w
        lane = lax.iota(jnp.int32, 16)

        for t, src in ((0, tgt_hbm), (1, ctx_hbm)):
            pltpu.sync_copy(src.at[pl.ds(base, b_per_w)], idx_v)
            for j in range(b_per_w // 16):
                va = idx_v[pl.ds(j * 16, 16)]
                for l in range(16):
                    idx_s[t * b_per_w + j * 16 + l] = jnp.sum(
                        jnp.where(lane == l, va, 0))

        def run_table(t, wt, out_hbm):
            def fire(i, slot, sem):
                v = idx_s[t * b_per_w + i]
                off = pl.multiple_of((v // 128) * 128, 128)
                pltpu.async_copy(wt.at[:, pl.ds(off, 128)],
                                 pnl.at[pl.ds(slot * D, D), :], sem)

            def extract(i, slot):
                v = idx_s[t * b_per_w + i]
                r = lax.rem(v, 128)
                coli = jnp.full((16,), r, jnp.int32)
                for c4 in range(D // 16):
                    rowi = slot * D + c4 * 16 + lane
                    vals = plsc.load_gather(pnl, [rowi, coli])
                    rows[pl.ds(i * 128 + c4 * 16, 16)] = vals

            n_grp = b_per_w // _GRP
            n_pairs = n_grp // 2
            for s in range(_GRP):
                fire(s, s, sem0)
            for s in range(_GRP):
                fire(_GRP + s, _GRP + s, sem1)

            def body(p, carry):
                for half, sem in ((0, sem0), (1, sem1)):
                    g = 2 * p + half
                    for s in range(_GRP):
                        pltpu.make_async_copy(
                            wt.at[:, pl.ds(0, 128)],
                            pnl.at[pl.ds(0, D), :], sem).wait()
                    for s in range(_GRP):
                        extract(g * _GRP + s, half * _GRP + s)

                    @pl.when(p < n_pairs - 1)
                    def _():
                        for s in range(_GRP):
                            fire((g + 2) * _GRP + s, half * _GRP + s, sem)
                return carry

            lax.fori_loop(0, n_pairs, body, 0)
            pltpu.sync_copy(rows, out_hbm.at[pl.ds(base * 128, b_per_w * 128)])

        run_table(0, wt_a, out_a)
        run_table(1, wt_b, out_b)

    return gather_k(in_w, out_w, target, context)


_TCHUNK = 128


def _tc_gather(wt, idx, start, count):
    D = wt.shape[0]
    T = count
    n_steps = T // _TCHUNK

    def body(idx_ref, wt_ref, o_ref, pnl, sem_a, sem_b):
        k = pl.program_id(0)
        lanes = lax.broadcasted_iota(jnp.int32, (8, 128), 1)
        half_sz = _TCHUNK * D

        def fire(chunk, half, sem):
            for j in range(_TCHUNK):
                v = idx_ref[start + chunk * _TCHUNK + j]
                off = pl.multiple_of((v // 128) * 128, 128)
                pltpu.make_async_copy(
                    wt_ref.at[:, pl.ds(off, 128)],
                    pnl.at[pl.ds(half * half_sz + j * D, D), :],
                    sem).start()

        def drain_extract(chunk, half, sem):
            pltpu.make_async_copy(
                wt_ref.at[:, pl.ds(0, 128 * _TCHUNK)],
                pnl.at[pl.ds(half * half_sz, half_sz), :], sem).wait()
            for b in range(_TCHUNK // 8):
                rs = [idx_ref[start + chunk * _TCHUNK + b * 8 + p] % 128
                      for p in range(8)]
                rcol = jnp.concatenate(
                    [jnp.full((1, 128), r, jnp.int32) for r in rs], axis=0)
                oh = (lanes == rcol).astype(jnp.float32)
                pstack = pnl[pl.ds(half * half_sz + b * 8 * D, 8 * D), :]
                m2 = lax.dot_general(oh, pstack, (((1,), (1,)), ((), ())),
                                     preferred_element_type=jnp.float32)
                rows8 = jnp.concatenate(
                    [lax.slice(m2, (p, p * D), (p + 1, p * D + D))
                     for p in range(8)], axis=0)
                o_ref[pl.ds(b * 8, 8), :] = rows8

        even = lax.rem(k, 2) == 0

        @pl.when(k == 0)
        def _():
            fire(0, 0, sem_a)

        @pl.when(jnp.logical_and(k + 1 < n_steps, even))
        def _():
            fire(k + 1, 1, sem_b)

        @pl.when(jnp.logical_and(k + 1 < n_steps,
                                 jnp.logical_not(even)))
        def _():
            fire(k + 1, 0, sem_a)

        @pl.when(even)
        def _():
            drain_extract(k, 0, sem_a)

        @pl.when(jnp.logical_not(even))
        def _():
            drain_extract(k, 1, sem_b)

    return pl.pallas_call(
        body,
        grid_spec=pltpu.PrefetchScalarGridSpec(
            num_scalar_prefetch=1,
            grid=(n_steps,),
            in_specs=[pl.BlockSpec(memory_space=pl.ANY)],
            out_specs=pl.BlockSpec((_TCHUNK, D), lambda k, idx: (k, 0)),
            scratch_shapes=[
                pltpu.VMEM((2 * _TCHUNK * D, 128), jnp.float32),
                pltpu.SemaphoreType.DMA,
                pltpu.SemaphoreType.DMA,
            ],
        ),
        out_shape=jax.ShapeDtypeStruct((T, D), jnp.float32),
    )(idx, wt)


def _tc_score_split(a_sc, a_tc, b_sc, b_tc):
    t, D = a_tc.shape
    s = a_sc.shape[0] // 128
    B = s + t
    BLK = 512
    n_sc = s // BLK

    def matmul_body(a_ref, at_ref, b_ref, bt_ref, o_ref):
        i = pl.program_id(0)
        dims = (((1,), (1,)), ((), ()))
        a_p = a_ref[...].reshape(BLK, 128)
        a = jnp.where(i < n_sc, lax.slice(a_p, (0, 0), (BLK, D)),
                      at_ref[...])
        b_p = b_ref[...].reshape(s, 128)
        o_ref[:, pl.ds(0, s)] = lax.dot_general(
            a, lax.slice(b_p, (0, 0), (s, D)), dims,
            preferred_element_type=jnp.float32)
        o_ref[:, pl.ds(s, t)] = lax.dot_general(
            a, bt_ref[...], dims, preferred_element_type=jnp.float32)

    return pl.pallas_call(
        matmul_body,
        grid=(B // BLK,),
        in_specs=[
            pl.BlockSpec((BLK * 128,),
                         lambda i: (jnp.minimum(i, n_sc - 1),)),
            pl.BlockSpec((BLK, D),
                         lambda i: (jnp.maximum(i - n_sc, 0), 0)),
            pl.BlockSpec((s * 128,), lambda i: (0,)),
            pl.BlockSpec((t, D), lambda i: (0, 0)),
        ],
        out_specs=pl.BlockSpec((BLK, B), lambda i: (i, 0)),
        out_shape=jax.ShapeDtypeStruct((B, B), jnp.float32),
    )(a_sc, a_tc, b_sc, b_tc)


_SC_SHARE = 2560


def kernel(target, context, in_embed_w, out_embed_w):
    target = target.astype(jnp.int32)
    context = context.astype(jnp.int32)
    D = in_embed_w.shape[1]
    wt_a = in_embed_w.T
    wt_b = out_embed_w.T
    s = _SC_SHARE
    t = target.shape[0] - s
    a_flat, b_flat = _sc_gather_pair(wt_a, wt_b, target, context, s)
    a_tc = _tc_gather(wt_a, target, s, t)
    b_tc = _tc_gather(wt_b, context, s, t)
    return _tc_score_split(a_flat, a_tc, b_flat, b_tc)


# speedup vs baseline: 1.1676x; 1.0617x over previous
import functools

import jax
import jax.numpy as jnp
from jax import lax
from jax.experimental import pallas as pl
from jax.experimental.pallas import tpu as pltpu
from jax.experimental.pallas import tpu_sc as plsc

_GRP = 4


def _sc_gather_pair(in_w, out_w, target, context, B):
    D, V = in_w.shape
    info = plsc.get_sparse_core_info()
    NC, NS = info.num_cores, info.num_subcores
    NW = NC * NS
    b_per_w = B // NW
    mesh = plsc.VectorSubcoreMesh(core_axis_name="c", subcore_axis_name="s")

    @functools.partial(
        pl.kernel,
        out_type=(
            jax.ShapeDtypeStruct((B * 128,), jnp.float32),
            jax.ShapeDtypeStruct((B * 128,), jnp.float32),
        ),
        mesh=mesh,
        compiler_params=pltpu.CompilerParams(needs_layout_passes=False),
        scratch_types=[
            pltpu.VMEM((b_per_w,), jnp.int32),
            pltpu.SMEM((2 * b_per_w,), jnp.int32),
            pltpu.VMEM((2 * _GRP * D, 128), jnp.float32),
            pltpu.VMEM((b_per_w * 128,), jnp.float32),
            pltpu.SemaphoreType.DMA,
            pltpu.SemaphoreType.DMA,
        ],
    )
    def gather_k(wt_a, wt_b, tgt_hbm, ctx_hbm, out_a, out_b,
                 idx_v, idx_s, pnl, rows, sem0, sem1):
        wid = lax.axis_index("s") * NC + lax.axis_index("c")
        base = wid * b_per_w
        lane = lax.iota(jnp.int32, 16)

        for t, src in ((0, tgt_hbm), (1, ctx_hbm)):
            pltpu.sync_copy(src.at[pl.ds(base, b_per_w)], idx_v)
            for j in range(b_per_w // 16):
                va = idx_v[pl.ds(j * 16, 16)]
                for l in range(16):
                    idx_s[t * b_per_w + j * 16 + l] = jnp.sum(
                        jnp.where(lane == l, va, 0))

        def run_table(t, wt, out_hbm):
            def fire(i, slot, sem):
                v = idx_s[t * b_per_w + i]
                off = pl.multiple_of((v // 128) * 128, 128)
                pltpu.async_copy(wt.at[:, pl.ds(off, 128)],
                                 pnl.at[pl.ds(slot * D, D), :], sem)

            def extract(i, slot):
                v = idx_s[t * b_per_w + i]
                r = lax.rem(v, 128)
                coli = jnp.full((16,), r, jnp.int32)
                for c4 in range(D // 16):
                    rowi = slot * D + c4 * 16 + lane
                    vals = plsc.load_gather(pnl, [rowi, coli])
                    rows[pl.ds(i * 128 + c4 * 16, 16)] = vals

            n_grp = b_per_w // _GRP
            n_pairs = n_grp // 2
            for s in range(_GRP):
                fire(s, s, sem0)
            for s in range(_GRP):
                fire(_GRP + s, _GRP + s, sem1)

            def body(p, carry):
                for half, sem in ((0, sem0), (1, sem1)):
                    g = 2 * p + half
                    for s in range(_GRP):
                        pltpu.make_async_copy(
                            wt.at[:, pl.ds(0, 128)],
                            pnl.at[pl.ds(0, D), :], sem).wait()
                    for s in range(_GRP):
                        extract(g * _GRP + s, half * _GRP + s)

                    @pl.when(p < n_pairs - 1)
                    def _():
                        for s in range(_GRP):
                            fire((g + 2) * _GRP + s, half * _GRP + s, sem)
                return carry

            lax.fori_loop(0, n_pairs, body, 0)
            pltpu.sync_copy(rows, out_hbm.at[pl.ds(base * 128, b_per_w * 128)])

        run_table(0, wt_a, out_a)
        run_table(1, wt_b, out_b)

    return gather_k(in_w, out_w, target, context)


_TCHUNK = 128


def _tc_gather(wt, idx, start, count):
    D = wt.shape[0]
    T = count
    n_steps = T // _TCHUNK

    def body(idx_ref, wt_ref, o_ref, pnl, sem_a, sem_b):
        k = pl.program_id(0)
        lanes = lax.broadcasted_iota(jnp.int32, (8, 128), 1)
        half_sz = _TCHUNK * D

        def fire(chunk, half, sem):
            for j in range(_TCHUNK):
                v = idx_ref[start + chunk * _TCHUNK + j]
                off = pl.multiple_of((v // 128) * 128, 128)
                pltpu.make_async_copy(
                    wt_ref.at[:, pl.ds(off, 128)],
                    pnl.at[pl.ds(half * half_sz + j * D, D), :],
                    sem).start()

        def drain_extract(chunk, half, sem):
            pltpu.make_async_copy(
                wt_ref.at[:, pl.ds(0, 128 * _TCHUNK)],
                pnl.at[pl.ds(half * half_sz, half_sz), :], sem).wait()
            for b in range(_TCHUNK // 8):
                rs = [idx_ref[start + chunk * _TCHUNK + b * 8 + p] % 128
                      for p in range(8)]
                rcol = jnp.concatenate(
                    [jnp.full((1, 128), r, jnp.int32) for r in rs], axis=0)
                oh = (lanes == rcol).astype(jnp.float32)
                pstack = pnl[pl.ds(half * half_sz + b * 8 * D, 8 * D), :]
                m2 = lax.dot_general(oh, pstack, (((1,), (1,)), ((), ())),
                                     preferred_element_type=jnp.float32)
                rows8 = jnp.concatenate(
                    [lax.slice(m2, (p, p * D), (p + 1, p * D + D))
                     for p in range(8)], axis=0)
                o_ref[pl.ds(b * 8, 8), :] = rows8

        even = lax.rem(k, 2) == 0

        @pl.when(k == 0)
        def _():
            fire(0, 0, sem_a)

        @pl.when(jnp.logical_and(k + 1 < n_steps, even))
        def _():
            fire(k + 1, 1, sem_b)

        @pl.when(jnp.logical_and(k + 1 < n_steps,
                                 jnp.logical_not(even)))
        def _():
            fire(k + 1, 0, sem_a)

        @pl.when(even)
        def _():
            drain_extract(k, 0, sem_a)

        @pl.when(jnp.logical_not(even))
        def _():
            drain_extract(k, 1, sem_b)

    return pl.pallas_call(
        body,
        grid_spec=pltpu.PrefetchScalarGridSpec(
            num_scalar_prefetch=1,
            grid=(n_steps,),
            in_specs=[pl.BlockSpec(memory_space=pl.ANY)],
            out_specs=pl.BlockSpec((_TCHUNK, D), lambda k, idx: (k, 0)),
            scratch_shapes=[
                pltpu.VMEM((2 * _TCHUNK * D, 128), jnp.float32),
                pltpu.SemaphoreType.DMA,
                pltpu.SemaphoreType.DMA,
            ],
        ),
        out_shape=jax.ShapeDtypeStruct((T, D), jnp.float32),
    )(idx, wt)


def _tc_score_split(a_sc, a_tc, b_sc, b_tc):
    t, D = a_tc.shape
    s = a_sc.shape[0] // 128
    B = s + t
    BLK = 512
    n_sc = s // BLK

    def matmul_body(a_ref, at_ref, b_ref, bt_ref, o_ref):
        i = pl.program_id(0)
        dims = (((1,), (1,)), ((), ()))
        a_p = a_ref[...].reshape(BLK, 128)
        a = jnp.where(i < n_sc, lax.slice(a_p, (0, 0), (BLK, D)),
                      at_ref[...])
        b_p = b_ref[...].reshape(s, 128)
        o_ref[:, pl.ds(0, s)] = lax.dot_general(
            a, lax.slice(b_p, (0, 0), (s, D)), dims,
            preferred_element_type=jnp.float32)
        o_ref[:, pl.ds(s, t)] = lax.dot_general(
            a, bt_ref[...], dims, preferred_element_type=jnp.float32)

    return pl.pallas_call(
        matmul_body,
        grid=(B // BLK,),
        in_specs=[
            pl.BlockSpec((BLK * 128,),
                         lambda i: (jnp.minimum(i, n_sc - 1),)),
            pl.BlockSpec((BLK, D),
                         lambda i: (jnp.maximum(i - n_sc, 0), 0)),
            pl.BlockSpec((s * 128,), lambda i: (0,)),
            pl.BlockSpec((t, D), lambda i: (0, 0)),
        ],
        out_specs=pl.BlockSpec((BLK, B), lambda i: (i, 0)),
        out_shape=jax.ShapeDtypeStruct((B, B), jnp.float32),
    )(a_sc, a_tc, b_sc, b_tc)


_SC_SHARE = 2048


def kernel(target, context, in_embed_w, out_embed_w):
    target = target.astype(jnp.int32)
    context = context.astype(jnp.int32)
    D = in_embed_w.shape[1]
    wt_a = in_embed_w.T
    wt_b = out_embed_w.T
    s = _SC_SHARE
    t = target.shape[0] - s
    a_flat, b_flat = _sc_gather_pair(wt_a, wt_b, target, context, s)
    a_tc = _tc_gather(wt_a, target, s, t)
    b_tc = _tc_gather(wt_b, context, s, t)
    return _tc_score_split(a_flat, a_tc, b_flat, b_tc)
